# Initial kernel scaffold; baseline (speedup 1.0000x reference)
import jax
import jax.numpy as jnp
from jax.experimental import pallas as pl


def kernel(x, w_shift, w_scale_log, w_proj, b_proj):
    raise NotImplementedError("write your pallas kernel here")


# speedup vs baseline: 94.8514x; 94.8514x over previous
import jax
import jax.numpy as jnp
from jax.experimental import pallas as pl
from jax.experimental.pallas import tpu as pltpu

MOMENTUM = 0.01
EPS = 1e-6
TBLK = 512


def _an_body(w_a_ref, At_ref, dec_ref, wp_ref, bp_ref, x_ref, o_ref, carry_ref):
    t = pl.program_id(1)

    @pl.when(t == 0)
    def _():
        carry_ref[0] = 0.0
        carry_ref[1] = 0.0

    xb = x_ref[0]
    su = jax.lax.dot_general(
        w_a_ref[:, :], xb, (((1,), (0,)), ((), ())),
        preferred_element_type=jnp.float32,
        precision=jax.lax.Precision.HIGHEST)
    v = jax.lax.dot_general(
        w_a_ref[1:2, :], xb * xb, (((1,), (0,)), ((), ())),
        preferred_element_type=jnp.float32,
        precision=jax.lax.Precision.HIGHEST)
    s = su[0:1, :]
    u = su[1:2, :]
    wsum = jnp.sum(w_a_ref[1:2, :])

    dec = dec_ref[0:1, :]
    shift = jax.lax.dot_general(
        s, At_ref[:, :], (((1,), (0,)), ((), ())),
        preferred_element_type=jnp.float32,
        precision=jax.lax.Precision.HIGHEST) + carry_ref[0] * dec
    q = v - 2.0 * shift * u + (shift * shift) * wsum
    scale = jax.lax.dot_general(
        q, At_ref[:, :], (((1,), (0,)), ((), ())),
        preferred_element_type=jnp.float32,
        precision=jax.lax.Precision.HIGHEST) + carry_ref[1] * dec

    carry_ref[0] = shift[0, TBLK - 1]
    carry_ref[1] = scale[0, TBLK - 1]

    inv = jax.lax.rsqrt(scale + EPS)
    o_ref[0] = (xb - shift) * inv * wp_ref[:, :] + bp_ref[:, :]


def kernel(x, w_shift, w_scale_log, w_proj, b_proj):
    B, C, T = x.shape
    nt = T // TBLK

    ew = jnp.exp(w_scale_log).astype(jnp.float32)
    w_a = jnp.zeros((8, C), jnp.float32).at[0].set(w_shift).at[1].set(ew)

    i = jnp.arange(TBLK)
    diff = i[None, :] - i[:, None]
    At = jnp.where(diff >= 0,
                   MOMENTUM * (1.0 - MOMENTUM) ** diff, 0.0).astype(jnp.float32)
    dec = ((1.0 - MOMENTUM) ** (i + 1)).astype(jnp.float32)[None, :]
    wp = w_proj.astype(jnp.float32)[:, None]
    bp = b_proj.astype(jnp.float32)[:, None]

    return pl.pallas_call(
        _an_body,
        grid=(B, nt),
        in_specs=[
            pl.BlockSpec((8, C), lambda b, t: (0, 0)),
            pl.BlockSpec((TBLK, TBLK), lambda b, t: (0, 0)),
            pl.BlockSpec((1, TBLK), lambda b, t: (0, 0)),
            pl.BlockSpec((C, 1), lambda b, t: (0, 0)),
            pl.BlockSpec((C, 1), lambda b, t: (0, 0)),
            pl.BlockSpec((1, C, TBLK), lambda b, t: (b, 0, t)),
        ],
        out_specs=pl.BlockSpec((1, C, TBLK), lambda b, t: (b, 0, t)),
        out_shape=jax.ShapeDtypeStruct((B, C, T), jnp.float32),
        scratch_shapes=[pltpu.SMEM((2,), jnp.float32)],
        compiler_params=pltpu.CompilerParams(
            dimension_semantics=("parallel", "arbitrary")),
    )(w_a, At, dec, wp, bp, x)


# speedup vs baseline: 172.8716x; 1.8226x over previous
import jax
import jax.numpy as jnp
from jax.experimental import pallas as pl
from jax.experimental.pallas import tpu as pltpu

MOMENTUM = 0.01
EPS = 1e-6
TBLK = 512


def _an_body(w_a_ref, At_ref, dec_ref, wp_ref, bp_ref, x_ref, o_ref, carry_ref):
    t = pl.program_id(1)

    @pl.when(t == 0)
    def _():
        carry_ref[0] = 0.0
        carry_ref[1] = 0.0

    xb = x_ref[0]
    su = jax.lax.dot_general(
        w_a_ref[:, :], xb, (((1,), (0,)), ((), ())),
        preferred_element_type=jnp.float32,
        precision=jax.lax.Precision.DEFAULT)
    v = jax.lax.dot_general(
        w_a_ref[1:2, :], xb * xb, (((1,), (0,)), ((), ())),
        preferred_element_type=jnp.float32,
        precision=jax.lax.Precision.DEFAULT)
    s = su[0:1, :]
    u = su[1:2, :]
    wsum = jnp.sum(w_a_ref[1:2, :])

    dec = dec_ref[0:1, :]
    shift = jax.lax.dot_general(
        s, At_ref[:, :], (((1,), (0,)), ((), ())),
        preferred_element_type=jnp.float32,
        precision=jax.lax.Precision.DEFAULT) + carry_ref[0] * dec
    q = v - 2.0 * shift * u + (shift * shift) * wsum
    scale = jax.lax.dot_general(
        q, At_ref[:, :], (((1,), (0,)), ((), ())),
        preferred_element_type=jnp.float32,
        precision=jax.lax.Precision.DEFAULT) + carry_ref[1] * dec

    carry_ref[0] = shift[0, TBLK - 1]
    carry_ref[1] = scale[0, TBLK - 1]

    inv = jax.lax.rsqrt(scale + EPS)
    o_ref[0] = (xb - shift) * inv * wp_ref[:, :] + bp_ref[:, :]


def kernel(x, w_shift, w_scale_log, w_proj, b_proj):
    B, C, T = x.shape
    nt = T // TBLK

    ew = jnp.exp(w_scale_log).astype(jnp.float32)
    w_a = jnp.zeros((8, C), jnp.float32).at[0].set(w_shift).at[1].set(ew)

    i = jnp.arange(TBLK)
    diff = i[None, :] - i[:, None]
    At = jnp.where(diff >= 0,
                   MOMENTUM * (1.0 - MOMENTUM) ** diff, 0.0).astype(jnp.float32)
    dec = ((1.0 - MOMENTUM) ** (i + 1)).astype(jnp.float32)[None, :]
    wp = w_proj.astype(jnp.float32)[:, None]
    bp = b_proj.astype(jnp.float32)[:, None]

    return pl.pallas_call(
        _an_body,
        grid=(B, nt),
        in_specs=[
            pl.BlockSpec((8, C), lambda b, t: (0, 0)),
            pl.BlockSpec((TBLK, TBLK), lambda b, t: (0, 0)),
            pl.BlockSpec((1, TBLK), lambda b, t: (0, 0)),
            pl.BlockSpec((C, 1), lambda b, t: (0, 0)),
            pl.BlockSpec((C, 1), lambda b, t: (0, 0)),
            pl.BlockSpec((1, C, TBLK), lambda b, t: (b, 0, t)),
        ],
        out_specs=pl.BlockSpec((1, C, TBLK), lambda b, t: (b, 0, t)),
        out_shape=jax.ShapeDtypeStruct((B, C, T), jnp.float32),
        scratch_shapes=[pltpu.SMEM((2,), jnp.float32)],
        compiler_params=pltpu.CompilerParams(
            dimension_semantics=("parallel", "arbitrary")),
    )(w_a, At, dec, wp, bp, x)


# speedup vs baseline: 341.3931x; 1.9748x over previous
import jax
import jax.numpy as jnp
from jax.experimental import pallas as pl
from jax.experimental.pallas import tpu as pltpu

MOMENTUM = 0.01
EPS = 1e-6
TBLK = 512
BGRP = 4


def _an_body(w_su_ref, w_v_ref, At_ref, dec_ref, wp_ref, bp_ref, x_ref, o_ref,
             carry_ref):
    t = pl.program_id(1)

    @pl.when(t == 0)
    def _():
        carry_ref[...] = jnp.zeros_like(carry_ref)

    xb = x_ref[...]
    c = xb.shape[1]
    xf = xb.reshape(BGRP * c, TBLK)

    su = jax.lax.dot_general(
        w_su_ref[...], xf, (((1,), (0,)), ((), ())),
        preferred_element_type=jnp.float32)
    v8 = jax.lax.dot_general(
        w_v_ref[...], xf * xf, (((1,), (0,)), ((), ())),
        preferred_element_type=jnp.float32)
    s = su[0:BGRP, :]
    u = su[BGRP:2 * BGRP, :]
    v = v8[0:BGRP, :]
    wsum = jnp.sum(w_v_ref[0:1, :])

    dec = dec_ref[0:1, :]
    shift = jax.lax.dot_general(
        s, At_ref[...], (((1,), (0,)), ((), ())),
        preferred_element_type=jnp.float32) + carry_ref[0:BGRP, 0:1] * dec
    q = v - 2.0 * shift * u + (shift * shift) * wsum
    scale = jax.lax.dot_general(
        q, At_ref[...], (((1,), (0,)), ((), ())),
        preferred_element_type=jnp.float32) + carry_ref[BGRP:2 * BGRP, 0:1] * dec

    carry_ref[0:BGRP, 0:1] = shift[:, TBLK - 1:TBLK]
    carry_ref[BGRP:2 * BGRP, 0:1] = scale[:, TBLK - 1:TBLK]

    inv = jax.lax.rsqrt(scale + EPS)
    o_ref[...] = ((xb - shift[:, None, :]) * inv[:, None, :]
                  * wp_ref[...][None, :, :] + bp_ref[...][None, :, :])


def kernel(x, w_shift, w_scale_log, w_proj, b_proj):
    B, C, T = x.shape
    nt = T // TBLK
    ng = B // BGRP

    ew = jnp.exp(w_scale_log).astype(jnp.float32)
    w_su = jnp.zeros((8, BGRP * C), jnp.float32)
    w_v = jnp.zeros((8, BGRP * C), jnp.float32)
    for i in range(BGRP):
        w_su = w_su.at[i, i * C:(i + 1) * C].set(w_shift)
        w_su = w_su.at[BGRP + i, i * C:(i + 1) * C].set(ew)
        w_v = w_v.at[i, i * C:(i + 1) * C].set(ew)

    i = jnp.arange(TBLK)
    diff = i[None, :] - i[:, None]
    At = jnp.where(diff >= 0,
                   MOMENTUM * (1.0 - MOMENTUM) ** diff, 0.0).astype(jnp.float32)
    dec = ((1.0 - MOMENTUM) ** (i + 1)).astype(jnp.float32)[None, :]
    wp = w_proj.astype(jnp.float32)[:, None]
    bp = b_proj.astype(jnp.float32)[:, None]

    return pl.pallas_call(
        _an_body,
        grid=(ng, nt),
        in_specs=[
            pl.BlockSpec((8, BGRP * C), lambda g, t: (0, 0)),
            pl.BlockSpec((8, BGRP * C), lambda g, t: (0, 0)),
            pl.BlockSpec((TBLK, TBLK), lambda g, t: (0, 0)),
            pl.BlockSpec((1, TBLK), lambda g, t: (0, 0)),
            pl.BlockSpec((C, 1), lambda g, t: (0, 0)),
            pl.BlockSpec((C, 1), lambda g, t: (0, 0)),
            pl.BlockSpec((BGRP, C, TBLK), lambda g, t: (g, 0, t)),
        ],
        out_specs=pl.BlockSpec((BGRP, C, TBLK), lambda g, t: (g, 0, t)),
        out_shape=jax.ShapeDtypeStruct((B, C, T), jnp.float32),
        scratch_shapes=[pltpu.VMEM((8, 128), jnp.float32)],
        compiler_params=pltpu.CompilerParams(
            dimension_semantics=("parallel", "arbitrary")),
    )(w_su, w_v, At, dec, wp, bp, x)
